# Initial kernel scaffold; baseline (speedup 1.0000x reference)
import jax
import jax.numpy as jnp
from jax.experimental import pallas as pl


def kernel(cosine, label):
    raise NotImplementedError("write your pallas kernel here")


# speedup vs baseline: 1.0592x; 1.0592x over previous
import functools

import jax
import jax.numpy as jnp
from jax.experimental import pallas as pl

S = 30.0
M = 0.4

_BLOCK_B = 256
_BLOCK_C = 4096


def _mcp_block(cosine_ref, label_ref, out_ref):
    j = pl.program_id(1)
    cols = jax.lax.broadcasted_iota(jnp.int32, cosine_ref.shape, 1) + j * _BLOCK_C
    mask = cols == label_ref[...]
    out_ref[...] = cosine_ref[...] * S - jnp.where(mask, S * M, 0.0)


@jax.jit
def kernel(cosine, label):
    B, C = cosine.shape
    label2d = label.astype(jnp.int32).reshape(B, 1)
    nb = pl.cdiv(B, _BLOCK_B)
    nc = pl.cdiv(C, _BLOCK_C)
    return pl.pallas_call(
        _mcp_block,
        grid=(nb, nc),
        in_specs=[
            pl.BlockSpec((_BLOCK_B, _BLOCK_C), lambda i, j: (i, j)),
            pl.BlockSpec((_BLOCK_B, 1), lambda i, j: (i, 0)),
        ],
        out_specs=pl.BlockSpec((_BLOCK_B, _BLOCK_C), lambda i, j: (i, j)),
        out_shape=jax.ShapeDtypeStruct((B, C), cosine.dtype),
    )(cosine, label2d)


# speedup vs baseline: 1.0652x; 1.0057x over previous
import functools

import jax
import jax.numpy as jnp
from jax.experimental import pallas as pl

S = 30.0
M = 0.4

_BLOCK_B = 256
_BLOCK_C = 8192


def _mcp_block(cosine_ref, label_ref, out_ref):
    j = pl.program_id(1)
    cols = jax.lax.broadcasted_iota(jnp.int32, cosine_ref.shape, 1) + j * _BLOCK_C
    mask = cols == label_ref[...]
    out_ref[...] = cosine_ref[...] * S - jnp.where(mask, S * M, 0.0)


@jax.jit
def kernel(cosine, label):
    B, C = cosine.shape
    label2d = label.astype(jnp.int32).reshape(B, 1)
    nb = pl.cdiv(B, _BLOCK_B)
    nc = pl.cdiv(C, _BLOCK_C)
    return pl.pallas_call(
        _mcp_block,
        grid=(nb, nc),
        in_specs=[
            pl.BlockSpec((_BLOCK_B, _BLOCK_C), lambda i, j: (i, j)),
            pl.BlockSpec((_BLOCK_B, 1), lambda i, j: (i, 0)),
        ],
        out_specs=pl.BlockSpec((_BLOCK_B, _BLOCK_C), lambda i, j: (i, j)),
        out_shape=jax.ShapeDtypeStruct((B, C), cosine.dtype),
    )(cosine, label2d)
